# Initial kernel scaffold; baseline (speedup 1.0000x reference)
import functools
import jax
import jax.numpy as jnp
from jax import lax
from jax.experimental import pallas as pl
from jax.experimental.pallas import tpu as pltpu
from jax.experimental.pallas import tpu_sc as plsc

B, L = 4096, 50
N = B * L
TOK_DIM, FEAT_DIM = 128, 32
OUT_DIM = TOK_DIM + 4 * FEAT_DIM

NC, NS = 2, 16
NW = NC * NS
RW = N // NW
G = 128
K = 2
S = K * G
CHUNKS = RW // S
IDROWS_PER_W = RW // G


def _body(tok_ids, pos_ids, shape_ids, prefix_ids, suffix_ids,
          tok_table, pos_table, shape_table, prefix_table, suffix_table,
          out,
          idx_tok, idx_pos, idx_shape, idx_prefix, idx_suffix,
          tok_rows, pos_rows, shape_rows, prefix_rows, suffix_rows,
          sem):
  c = lax.axis_index("c")
  s = lax.axis_index("s")
  wid = s * NC + c
  id_base = wid * IDROWS_PER_W

  def chunk(i, carry):
    idrow = id_base + i * K
    pltpu.sync_copy(tok_ids.at[pl.ds(idrow, K)], idx_tok)
    pltpu.sync_copy(pos_ids.at[pl.ds(idrow, K)], idx_pos)
    pltpu.sync_copy(shape_ids.at[pl.ds(idrow, K)], idx_shape)
    pltpu.sync_copy(prefix_ids.at[pl.ds(idrow, K)], idx_prefix)
    pltpu.sync_copy(suffix_ids.at[pl.ds(idrow, K)], idx_suffix)
    descs = []
    for j in range(K):
      dst = pl.ds(j * G, G)
      descs.append(pltpu.async_copy(tok_table.at[idx_tok.at[j]], tok_rows.at[dst], sem))
      descs.append(pltpu.async_copy(pos_table.at[idx_pos.at[j]], pos_rows.at[dst], sem))
      descs.append(pltpu.async_copy(shape_table.at[idx_shape.at[j]], shape_rows.at[dst], sem))
      descs.append(pltpu.async_copy(prefix_table.at[idx_prefix.at[j]], prefix_rows.at[dst], sem))
      descs.append(pltpu.async_copy(suffix_table.at[idx_suffix.at[j]], suffix_rows.at[dst], sem))
    for d in descs:
      d.wait()
    row0 = idrow * G
    pltpu.sync_copy(tok_rows, out.at[pl.ds(row0, S), pl.ds(0, TOK_DIM)])
    pltpu.sync_copy(pos_rows, out.at[pl.ds(row0, S), pl.ds(TOK_DIM, FEAT_DIM)])
    pltpu.sync_copy(shape_rows, out.at[pl.ds(row0, S), pl.ds(TOK_DIM + FEAT_DIM, FEAT_DIM)])
    pltpu.sync_copy(prefix_rows, out.at[pl.ds(row0, S), pl.ds(TOK_DIM + 2 * FEAT_DIM, FEAT_DIM)])
    pltpu.sync_copy(suffix_rows, out.at[pl.ds(row0, S), pl.ds(TOK_DIM + 3 * FEAT_DIM, FEAT_DIM)])
    return carry

  lax.fori_loop(0, CHUNKS, chunk, 0)


@jax.jit
def _embed(tok_ids, pos_ids, shape_ids, prefix_ids, suffix_ids,
           tok_table, pos_table, shape_table, prefix_table, suffix_table):
  mesh = plsc.VectorSubcoreMesh(core_axis_name="c", subcore_axis_name="s")
  run = pl.kernel(
      _body,
      out_type=jax.ShapeDtypeStruct((N, OUT_DIM), jnp.float32),
      mesh=mesh,
      scratch_types=[
          pltpu.VMEM((K, G), jnp.int32),
          pltpu.VMEM((K, G), jnp.int32),
          pltpu.VMEM((K, G), jnp.int32),
          pltpu.VMEM((K, G), jnp.int32),
          pltpu.VMEM((K, G), jnp.int32),
          pltpu.VMEM((S, TOK_DIM), jnp.float32),
          pltpu.VMEM((S, FEAT_DIM), jnp.float32),
          pltpu.VMEM((S, FEAT_DIM), jnp.float32),
          pltpu.VMEM((S, FEAT_DIM), jnp.float32),
          pltpu.VMEM((S, FEAT_DIM), jnp.float32),
          pltpu.SemaphoreType.DMA,
      ],
  )
  return run(tok_ids, pos_ids, shape_ids, prefix_ids, suffix_ids,
             tok_table, pos_table, shape_table, prefix_table, suffix_table)


def kernel(tok_ids, feat_ids_pos, feat_ids_shape, feat_ids_prefix, feat_ids_suffix,
           tok_table, pos_table, shape_table, prefix_table, suffix_table):
  ids2d = lambda x: x.astype(jnp.int32).reshape(N // G, G)
  out = _embed(ids2d(tok_ids), ids2d(feat_ids_pos), ids2d(feat_ids_shape),
               ids2d(feat_ids_prefix), ids2d(feat_ids_suffix),
               tok_table, pos_table, shape_table, prefix_table, suffix_table)
  return out.reshape(B, L, OUT_DIM)


# speedup vs baseline: 7.5307x; 7.5307x over previous
import functools
import jax
import jax.numpy as jnp
from jax import lax
from jax.experimental import pallas as pl
from jax.experimental.pallas import tpu as pltpu
from jax.experimental.pallas import tpu_sc as plsc

B, L = 4096, 50
N = B * L
TOK_DIM, FEAT_DIM = 128, 32
OUT_DIM = TOK_DIM + 4 * FEAT_DIM

NC, NS = 2, 16
NW = NC * NS
RW = N // NW
G = 128
K = 2
S = K * G
CHUNKS = RW // S
IDROWS_PER_W = RW // G


def _body(tok_ids, pos_ids, shape_ids, prefix_ids, suffix_ids,
          tok_table, pos_table, shape_table, prefix_table, suffix_table,
          out,
          idx_tok, idx_pos, idx_shape, idx_prefix, idx_suffix,
          tok_rows, pos_rows, shape_rows, prefix_rows, suffix_rows,
          sem):
  c = lax.axis_index("c")
  s = lax.axis_index("s")
  wid = s * NC + c
  id_base = wid * IDROWS_PER_W

  def chunk(i, carry):
    idrow = id_base + i * K
    pltpu.sync_copy(tok_ids.at[pl.ds(idrow, K)], idx_tok)
    pltpu.sync_copy(pos_ids.at[pl.ds(idrow, K)], idx_pos)
    pltpu.sync_copy(shape_ids.at[pl.ds(idrow, K)], idx_shape)
    pltpu.sync_copy(prefix_ids.at[pl.ds(idrow, K)], idx_prefix)
    pltpu.sync_copy(suffix_ids.at[pl.ds(idrow, K)], idx_suffix)
    descs = []
    for j in range(K):
      dst = pl.ds(j * G, G)
      descs.append(pltpu.async_copy(tok_table.at[idx_tok.at[j]], tok_rows.at[dst], sem))
      descs.append(pltpu.async_copy(pos_table.at[idx_pos.at[j]], pos_rows.at[dst], sem))
      descs.append(pltpu.async_copy(shape_table.at[idx_shape.at[j]], shape_rows.at[dst], sem))
      descs.append(pltpu.async_copy(prefix_table.at[idx_prefix.at[j]], prefix_rows.at[dst], sem))
      descs.append(pltpu.async_copy(suffix_table.at[idx_suffix.at[j]], suffix_rows.at[dst], sem))
    for d in descs:
      d.wait()
    row0 = idrow * G
    pltpu.sync_copy(tok_rows, out.at[pl.ds(row0, S), pl.ds(0, TOK_DIM)])
    pltpu.sync_copy(pos_rows, out.at[pl.ds(row0, S), pl.ds(TOK_DIM, FEAT_DIM)])
    pltpu.sync_copy(shape_rows, out.at[pl.ds(row0, S), pl.ds(TOK_DIM + FEAT_DIM, FEAT_DIM)])
    pltpu.sync_copy(prefix_rows, out.at[pl.ds(row0, S), pl.ds(TOK_DIM + 2 * FEAT_DIM, FEAT_DIM)])
    pltpu.sync_copy(suffix_rows, out.at[pl.ds(row0, S), pl.ds(TOK_DIM + 3 * FEAT_DIM, FEAT_DIM)])
    return carry

  lax.fori_loop(0, CHUNKS, chunk, 0)


@jax.jit
def _embed(tok_ids, pos_ids, shape_ids, prefix_ids, suffix_ids,
           tok_table, pos_table, shape_table, prefix_table, suffix_table):
  mesh = plsc.VectorSubcoreMesh(core_axis_name="c", subcore_axis_name="s")
  run = pl.kernel(
      _body,
      out_type=jax.ShapeDtypeStruct((N, OUT_DIM), jnp.float32),
      mesh=mesh,
      compiler_params=pltpu.CompilerParams(use_tc_tiling_on_sc=False),
      scratch_types=[
          pltpu.VMEM((K, G), jnp.int32),
          pltpu.VMEM((K, G), jnp.int32),
          pltpu.VMEM((K, G), jnp.int32),
          pltpu.VMEM((K, G), jnp.int32),
          pltpu.VMEM((K, G), jnp.int32),
          pltpu.VMEM((S, TOK_DIM), jnp.float32),
          pltpu.VMEM((S, FEAT_DIM), jnp.float32),
          pltpu.VMEM((S, FEAT_DIM), jnp.float32),
          pltpu.VMEM((S, FEAT_DIM), jnp.float32),
          pltpu.VMEM((S, FEAT_DIM), jnp.float32),
          pltpu.SemaphoreType.DMA,
      ],
  )
  return run(tok_ids, pos_ids, shape_ids, prefix_ids, suffix_ids,
             tok_table, pos_table, shape_table, prefix_table, suffix_table)


def kernel(tok_ids, feat_ids_pos, feat_ids_shape, feat_ids_prefix, feat_ids_suffix,
           tok_table, pos_table, shape_table, prefix_table, suffix_table):
  ids2d = lambda x: x.astype(jnp.int32).reshape(N // G, G)
  out = _embed(ids2d(tok_ids), ids2d(feat_ids_pos), ids2d(feat_ids_shape),
               ids2d(feat_ids_prefix), ids2d(feat_ids_suffix),
               tok_table, pos_table, shape_table, prefix_table, suffix_table)
  return out.reshape(B, L, OUT_DIM)


# speedup vs baseline: 7.5577x; 1.0036x over previous
import functools
import jax
import jax.numpy as jnp
from jax import lax
from jax.experimental import pallas as pl
from jax.experimental.pallas import tpu as pltpu
from jax.experimental.pallas import tpu_sc as plsc

B, L = 4096, 50
N = B * L
TOK_DIM, FEAT_DIM = 128, 32
OUT_DIM = TOK_DIM + 4 * FEAT_DIM
COLS = (0, TOK_DIM, TOK_DIM + FEAT_DIM, TOK_DIM + 2 * FEAT_DIM, TOK_DIM + 3 * FEAT_DIM)
DIMS = (TOK_DIM, FEAT_DIM, FEAT_DIM, FEAT_DIM, FEAT_DIM)

NC, NS = 2, 16
NW = NC * NS
RW = N // NW
G = 128
CHUNKS = RW // G
NBUF = 3
STEADY = (CHUNKS - 2) // NBUF


def _body(ids_all, tok_table, pos_table, shape_table, prefix_table, suffix_table,
          out, *scratch):
  c = lax.axis_index("c")
  s = lax.axis_index("s")
  wid = s * NC + c
  chunk_base = wid * CHUNKS
  tables = (tok_table, pos_table, shape_table, prefix_table, suffix_table)

  idx = scratch[0:NBUF]
  rows = [scratch[NBUF + 5 * b: NBUF + 5 * (b + 1)] for b in range(NBUF)]
  isem = scratch[6 * NBUF + 0]
  gsem = scratch[6 * NBUF + 1]
  wsem = scratch[6 * NBUF + 2]

  def idx_copy(chunk, b):
    return pltpu.make_async_copy(ids_all.at[chunk_base + chunk], idx[b], isem)

  def g_copies(b):
    return [pltpu.make_async_copy(tables[t].at[idx[b].at[t]], rows[b][t], gsem)
            for t in range(5)]

  def w_copies(chunk, b):
    row0 = (chunk_base + chunk) * G
    return [pltpu.make_async_copy(
        rows[b][t], out.at[pl.ds(row0, G), pl.ds(COLS[t], DIMS[t])], wsem)
            for t in range(5)]

  def issue(descs):
    for d in (descs if isinstance(descs, list) else [descs]):
      d.start()

  def drain(descs):
    for d in (descs if isinstance(descs, list) else [descs]):
      d.wait()

  issue(idx_copy(0, 0))
  drain(idx_copy(0, 0))
  issue(g_copies(0))
  issue(idx_copy(1, 1))

  def outer(io, carry):
    for k in range(NBUF):
      v = 2 + NBUF * io + k
      sw, sg, si = k, (k + 1) % NBUF, (k + 2) % NBUF
      drain(g_copies(sw))
      issue(w_copies(v - 2, sw))
      drain(idx_copy(v - 1, sg))
      issue(g_copies(sg))
      if k == 0:
        @pl.when(io > 0)
        def _():
          drain(w_copies(v - NBUF, si))
      else:
        drain(w_copies(v - NBUF, si))
      issue(idx_copy(v, si))
    return carry

  lax.fori_loop(0, STEADY, outer, 0)

  vend = 2 + NBUF * STEADY
  c2, c1 = vend - 2, vend - 1
  drain(g_copies(c2 % NBUF))
  issue(w_copies(c2, c2 % NBUF))
  drain(idx_copy(c1, c1 % NBUF))
  issue(g_copies(c1 % NBUF))
  drain(g_copies(c1 % NBUF))
  issue(w_copies(c1, c1 % NBUF))
  drain(w_copies(vend - 3, (vend - 3) % NBUF))
  drain(w_copies(c2, c2 % NBUF))
  drain(w_copies(c1, c1 % NBUF))


@jax.jit
def _embed(ids_all, tok_table, pos_table, shape_table, prefix_table, suffix_table):
  mesh = plsc.VectorSubcoreMesh(core_axis_name="c", subcore_axis_name="s")
  scratch = []
  for _ in range(NBUF):
    scratch.append(pltpu.VMEM((5, G), jnp.int32))
  for _ in range(NBUF):
    for d in DIMS:
      scratch.append(pltpu.VMEM((G, d), jnp.float32))
  scratch += [pltpu.SemaphoreType.DMA] * 3
  run = pl.kernel(
      _body,
      out_type=jax.ShapeDtypeStruct((N, OUT_DIM), jnp.float32),
      mesh=mesh,
      compiler_params=pltpu.CompilerParams(use_tc_tiling_on_sc=False),
      scratch_types=scratch,
  )
  return run(ids_all, tok_table, pos_table, shape_table, prefix_table, suffix_table)


def kernel(tok_ids, feat_ids_pos, feat_ids_shape, feat_ids_prefix, feat_ids_suffix,
           tok_table, pos_table, shape_table, prefix_table, suffix_table):
  ids2d = lambda x: x.astype(jnp.int32).reshape(N // G, G)
  ids_all = jnp.stack([ids2d(tok_ids), ids2d(feat_ids_pos), ids2d(feat_ids_shape),
                       ids2d(feat_ids_prefix), ids2d(feat_ids_suffix)], axis=1)
  out = _embed(ids_all, tok_table, pos_table, shape_table, prefix_table, suffix_table)
  return out.reshape(B, L, OUT_DIM)


# speedup vs baseline: 8.7132x; 1.1529x over previous
import functools
import jax
import jax.numpy as jnp
from jax import lax
from jax.experimental import pallas as pl
from jax.experimental.pallas import tpu as pltpu
from jax.experimental.pallas import tpu_sc as plsc

B, L = 4096, 50
N = B * L
TOK_DIM, FEAT_DIM = 128, 32
OUT_DIM = TOK_DIM + 4 * FEAT_DIM

NC, NS = 2, 16
NW = NC * NS
BPW = B // NW
CB = 2
R = CB * L
CHUNKS = BPW // CB
GRP = 8
GROUPS = CHUNKS // GRP


def _body(tok2d, fidx2d, tok_table, f0, f1, f2, f3, out,
          itok, ifeat, tokbuf, comb, gsem):
  c = lax.axis_index("c")
  s = lax.axis_index("s")
  wid = s * NC + c
  batch0 = wid * BPW
  ftabs = (f0, f1, f2, f3)

  def group(g, carry):
    grow = wid * CHUNKS + g * GRP
    pltpu.sync_copy(tok2d.at[pl.ds(grow, GRP)], itok)
    pltpu.sync_copy(fidx2d.at[pl.ds(4 * grow, 4 * GRP)], ifeat)
    for m in range(GRP):
      pltpu.async_copy(f0.at[ifeat.at[4 * m]], comb, gsem).wait()
      dts = [pltpu.async_copy(tok_table.at[itok.at[m]], tokbuf, gsem)]
      for t in range(1, 4):
        dts.append(pltpu.async_copy(ftabs[t].at[ifeat.at[4 * m + t]], comb, gsem,
                                    add=True))
      for d in dts:
        d.wait()
      b0 = batch0 + (g * GRP + m) * CB
      pltpu.sync_copy(tokbuf.reshape(CB, L, TOK_DIM),
                      out.at[pl.ds(b0, CB), :, pl.ds(0, TOK_DIM)])
      pltpu.sync_copy(comb.reshape(CB, L, TOK_DIM),
                      out.at[pl.ds(b0, CB), :, pl.ds(TOK_DIM, TOK_DIM)])
    return carry

  lax.fori_loop(0, GROUPS, group, 0)


@jax.jit
def _embed(tok2d, fidx2d, tok_table, f0, f1, f2, f3):
  mesh = plsc.VectorSubcoreMesh(core_axis_name="c", subcore_axis_name="s")
  run = pl.kernel(
      _body,
      out_type=jax.ShapeDtypeStruct((B, L, OUT_DIM), jnp.float32),
      mesh=mesh,
      scratch_types=[
          pltpu.VMEM((GRP, R), jnp.int32),
          pltpu.VMEM((4 * GRP, R), jnp.int32),
          pltpu.VMEM((R, TOK_DIM), jnp.float32),
          pltpu.VMEM((R, TOK_DIM), jnp.float32),
          pltpu.SemaphoreType.DMA,
      ],
  )
  return run(tok2d, fidx2d, tok_table, f0, f1, f2, f3)


def _shift_pad(table, quarter):
  v = table.shape[0]
  padded = jnp.zeros((v, TOK_DIM), jnp.float32)
  return lax.dynamic_update_slice(padded, table, (0, quarter * FEAT_DIM))


def kernel(tok_ids, feat_ids_pos, feat_ids_shape, feat_ids_prefix, feat_ids_suffix,
           tok_table, pos_table, shape_table, prefix_table, suffix_table):
  tok2d = tok_ids.astype(jnp.int32).reshape(N // R, R)
  fidx = jnp.stack([feat_ids_pos.astype(jnp.int32).reshape(N // R, R),
                    feat_ids_shape.astype(jnp.int32).reshape(N // R, R),
                    feat_ids_prefix.astype(jnp.int32).reshape(N // R, R),
                    feat_ids_suffix.astype(jnp.int32).reshape(N // R, R)],
                   axis=1).reshape(4 * N // R, R)
  return _embed(tok2d, fidx, tok_table,
                _shift_pad(pos_table, 0), _shift_pad(shape_table, 1),
                _shift_pad(prefix_table, 2), _shift_pad(suffix_table, 3))


# speedup vs baseline: 14.5811x; 1.6735x over previous
import functools
import jax
import jax.numpy as jnp
from jax import lax
from jax.experimental import pallas as pl
from jax.experimental.pallas import tpu as pltpu
from jax.experimental.pallas import tpu_sc as plsc

B, L = 4096, 50
N = B * L
TOK_DIM, FEAT_DIM = 128, 32
OUT_DIM = TOK_DIM + 4 * FEAT_DIM

NC, NS = 2, 16
NW = NC * NS
G = 128
NL, NB = 2, 64
LT, BT = L // NL, B // NB
NCHUNK = LT * BT
CPW = NCHUNK // NW
NS3 = 3


def _body(tok_flat, feat_flat, tok_table, f0, f1, f2, f3, out, *scr):
  c = lax.axis_index("c")
  s = lax.axis_index("s")
  wid = s * NC + c
  t0 = wid * CPW
  ftabs = (f0, f1, f2, f3)

  itok = scr[0:NS3]
  ifeat = scr[NS3:2 * NS3]
  tokbuf = scr[2 * NS3:3 * NS3]
  comb = scr[3 * NS3:4 * NS3]
  isem = scr[4 * NS3:5 * NS3]
  psem = scr[5 * NS3:6 * NS3]
  gsem = scr[6 * NS3:7 * NS3]
  wsem = scr[7 * NS3:8 * NS3]

  def idx_copies(ch, b):
    t = t0 + ch
    return [pltpu.make_async_copy(tok_flat.at[pl.ds(G * t, G)], itok[b], isem[b]),
            pltpu.make_async_copy(feat_flat.at[pl.ds(4 * G * t, 4 * G)], ifeat[b], isem[b])]

  def pos_copy(b):
    return pltpu.make_async_copy(f0.at[ifeat[b].at[pl.ds(0, G)]], comb[b], psem[b])

  def main_copies(b):
    dts = [pltpu.make_async_copy(tok_table.at[itok[b]], tokbuf[b], gsem[b])]
    for q in range(1, 4):
      dts.append(pltpu.make_async_copy(
          ftabs[q].at[ifeat[b].at[pl.ds(q * G, G)]], comb[b], gsem[b]))
    return dts

  def w_copies(ch, b):
    t = t0 + ch
    lt = lax.rem(t, LT)
    bt = lax.div(t, LT)
    dst0 = out.at[pl.ds(NL * lt, NL), pl.ds(NB * bt, NB), pl.ds(0, TOK_DIM)]
    dst1 = out.at[pl.ds(NL * lt, NL), pl.ds(NB * bt, NB), pl.ds(TOK_DIM, TOK_DIM)]
    return [pltpu.make_async_copy(tokbuf[b].reshape(NL, NB, TOK_DIM), dst0, wsem[b]),
            pltpu.make_async_copy(comb[b].reshape(NL, NB, TOK_DIM), dst1, wsem[b])]

  def issue(ds_):
    for d in ds_:
      d.start()

  def drain(ds_):
    for d in ds_:
      d.wait()

  issue(idx_copies(0, 0))

  def outer(io, carry):
    for k in range(NS3):
      v = NS3 * io + k
      sa, sb, sw, si = k, (k + 2) % NS3, (k + 1) % NS3, (k + 1) % NS3

      @pl.when(v <= CPW - 1)
      def _ga():
        @pl.when(v >= NS3)
        def _():
          drain(w_copies(v - NS3, sa))
        drain(idx_copies(v, sa))
        pos_copy(sa).start()
        issue(main_copies(sa)[:1])

      @pl.when(jnp.logical_and(v >= 1, v <= CPW))
      def _gb():
        pos_copy(sb).wait()
        for d in main_copies(sb)[1:]:
          d.start(add=True)

      @pl.when(jnp.logical_and(v >= 2, v <= CPW + 1))
      def _w():
        drain(main_copies(sw))
        issue(w_copies(v - 2, sw))

      @pl.when(v + 1 <= CPW - 1)
      def _i():
        issue(idx_copies(v + 1, si))
    return carry

  lax.fori_loop(0, (CPW + NS3 + 1) // NS3, outer, 0)

  for ch in range(CPW - NS3, CPW):
    drain(w_copies(ch, ch % NS3))


@jax.jit
def _embed(tok_flat, feat_flat, tok_table, f0, f1, f2, f3):
  mesh = plsc.VectorSubcoreMesh(core_axis_name="c", subcore_axis_name="s")
  scr = []
  scr += [pltpu.VMEM((G,), jnp.int32)] * NS3
  scr += [pltpu.VMEM((4 * G,), jnp.int32)] * NS3
  scr += [pltpu.VMEM((G, TOK_DIM), jnp.float32)] * NS3
  scr += [pltpu.VMEM((G, TOK_DIM), jnp.float32)] * NS3
  scr += [pltpu.SemaphoreType.DMA] * (4 * NS3)
  run = pl.kernel(
      _body,
      out_type=jax.ShapeDtypeStruct((L, B, OUT_DIM), jnp.float32),
      mesh=mesh,
      scratch_types=scr,
  )
  return run(tok_flat, feat_flat, tok_table, f0, f1, f2, f3)


def _shift_pad(table, quarter):
  v = table.shape[0]
  padded = jnp.zeros((v, TOK_DIM), jnp.float32)
  return lax.dynamic_update_slice(padded, table, (0, quarter * FEAT_DIM))


def _perm(ids):
  return ids.astype(jnp.int32).reshape(BT, NB, LT, NL).transpose(0, 2, 3, 1)


def kernel(tok_ids, feat_ids_pos, feat_ids_shape, feat_ids_prefix, feat_ids_suffix,
           tok_table, pos_table, shape_table, prefix_table, suffix_table):
  tok_flat = _perm(tok_ids).reshape(-1)
  feat_flat = jnp.stack([_perm(feat_ids_pos), _perm(feat_ids_shape),
                         _perm(feat_ids_prefix), _perm(feat_ids_suffix)],
                        axis=2).reshape(-1)
  outT = _embed(tok_flat, feat_flat, tok_table,
                _shift_pad(pos_table, 0), _shift_pad(shape_table, 1),
                _shift_pad(prefix_table, 2), _shift_pad(suffix_table, 3))
  return outT.transpose(1, 0, 2)


# speedup vs baseline: 15.6236x; 1.0715x over previous
import functools
import jax
import jax.numpy as jnp
from jax import lax
from jax.experimental import pallas as pl
from jax.experimental.pallas import tpu as pltpu
from jax.experimental.pallas import tpu_sc as plsc

B, L = 4096, 50
N = B * L
TOK_DIM, FEAT_DIM = 128, 32
OUT_DIM = TOK_DIM + 4 * FEAT_DIM

NC, NS = 2, 16
NW = NC * NS
G = 128
CPW = L
NS3 = 3


def _body(tokT, featT, tok_table, f0, f1, f2, f3, out, *scr):
  c = lax.axis_index("c")
  s = lax.axis_index("s")
  wid = s * NC + c
  b0 = wid * G
  ftabs = (f0, f1, f2, f3)

  itok = scr[0:NS3]
  ifeat = scr[NS3:2 * NS3]
  tokbuf = scr[2 * NS3:3 * NS3]
  comb = scr[3 * NS3:4 * NS3]
  isem = scr[4 * NS3:5 * NS3]
  psem = scr[5 * NS3:6 * NS3]
  gsem = scr[6 * NS3:7 * NS3]
  wsem = scr[7 * NS3:8 * NS3]

  def idx_copies(l, b):
    return [pltpu.make_async_copy(tokT.at[pl.ds(l, 1), pl.ds(b0, G)], itok[b], isem[b]),
            pltpu.make_async_copy(featT.at[l, :, pl.ds(b0, G)], ifeat[b], isem[b])]

  def pos_copy(b):
    return pltpu.make_async_copy(f0.at[ifeat[b].at[0]], comb[b], psem[b])

  def main_copies(b):
    dts = [pltpu.make_async_copy(tok_table.at[itok[b].at[0]], tokbuf[b], gsem[b])]
    for q in range(1, 4):
      dts.append(pltpu.make_async_copy(
          ftabs[q].at[ifeat[b].at[q]], comb[b], gsem[b]))
    return dts

  def w_copies(l, b):
    dst0 = out.at[pl.ds(l, 1), pl.ds(b0, G), pl.ds(0, TOK_DIM)]
    dst1 = out.at[pl.ds(l, 1), pl.ds(b0, G), pl.ds(TOK_DIM, TOK_DIM)]
    return [pltpu.make_async_copy(tokbuf[b].reshape(1, G, TOK_DIM), dst0, wsem[b]),
            pltpu.make_async_copy(comb[b].reshape(1, G, TOK_DIM), dst1, wsem[b])]

  def issue(ds_):
    for d in ds_:
      d.start()

  def drain(ds_):
    for d in ds_:
      d.wait()

  issue(idx_copies(0, 0))

  def outer(io, carry):
    for k in range(NS3):
      v = NS3 * io + k
      sa, sb, sw, si = k, (k + 2) % NS3, (k + 1) % NS3, (k + 1) % NS3

      @pl.when(v <= CPW - 1)
      def _ga():
        @pl.when(v >= NS3)
        def _():
          drain(w_copies(v - NS3, sa))
        drain(idx_copies(v, sa))
        pos_copy(sa).start()
        issue(main_copies(sa)[:1])

      @pl.when(jnp.logical_and(v >= 1, v <= CPW))
      def _gb():
        pos_copy(sb).wait()
        for d in main_copies(sb)[1:]:
          d.start(add=True)

      @pl.when(jnp.logical_and(v >= 2, v <= CPW + 1))
      def _w():
        drain(main_copies(sw))
        issue(w_copies(v - 2, sw))

      @pl.when(v + 1 <= CPW - 1)
      def _i():
        issue(idx_copies(v + 1, si))
    return carry

  lax.fori_loop(0, (CPW + NS3 + 1) // NS3, outer, 0)

  for ch in range(CPW - NS3, CPW):
    drain(w_copies(ch, ch % NS3))


@jax.jit
def _embed(tokT, featT, tok_table, f0, f1, f2, f3):
  mesh = plsc.VectorSubcoreMesh(core_axis_name="c", subcore_axis_name="s")
  scr = []
  scr += [pltpu.VMEM((1, G), jnp.int32)] * NS3
  scr += [pltpu.VMEM((4, G), jnp.int32)] * NS3
  scr += [pltpu.VMEM((G, TOK_DIM), jnp.float32)] * NS3
  scr += [pltpu.VMEM((G, TOK_DIM), jnp.float32)] * NS3
  scr += [pltpu.SemaphoreType.DMA] * (4 * NS3)
  run = pl.kernel(
      _body,
      out_type=jax.ShapeDtypeStruct((L, B, OUT_DIM), jnp.float32),
      mesh=mesh,
      scratch_types=scr,
  )
  return run(tokT, featT, tok_table, f0, f1, f2, f3)


def _shift_pad(table, quarter):
  v = table.shape[0]
  padded = jnp.zeros((v, TOK_DIM), jnp.float32)
  return lax.dynamic_update_slice(padded, table, (0, quarter * FEAT_DIM))


def kernel(tok_ids, feat_ids_pos, feat_ids_shape, feat_ids_prefix, feat_ids_suffix,
           tok_table, pos_table, shape_table, prefix_table, suffix_table):
  tokT = tok_ids.astype(jnp.int32).T
  featT = jnp.stack([feat_ids_pos.astype(jnp.int32).T,
                     feat_ids_shape.astype(jnp.int32).T,
                     feat_ids_prefix.astype(jnp.int32).T,
                     feat_ids_suffix.astype(jnp.int32).T], axis=1)
  outT = _embed(tokT, featT, tok_table,
                _shift_pad(pos_table, 0), _shift_pad(shape_table, 1),
                _shift_pad(prefix_table, 2), _shift_pad(suffix_table, 3))
  return outT.transpose(1, 0, 2)
